# baseline (speedup 1.0000x reference)
import functools

import jax
import jax.numpy as jnp
from jax import lax
from jax.experimental import pallas as pl
from jax.experimental.pallas import tpu as pltpu
from jax.experimental.pallas import tpu_sc as plsc

N = 2048
DIM = 128
K = 30
EI = 2 * DIM + 1
HID = 2 * EI
MDIM = 16
TOPK_BLK = 256
N_WORKERS = 32
ROWS_PER_W = (N * K) // N_WORKERS
GCH = 128
NCH = ROWS_PER_W // GCH


def _silu(x):
    return x * jax.nn.sigmoid(x)


def _topk_body(coords_ref, coords_t_ref, idx_ref, d_ref):
    dist = jnp.zeros((TOPK_BLK, N), jnp.float32)
    for c in range(3):
        col = coords_ref[:, c : c + 1]
        row = coords_t_ref[c : c + 1, :]
        diff = col - row
        dist = dist + diff * diff
    col_iota = lax.broadcasted_iota(jnp.int32, (TOPK_BLK, N), 1)
    sel_iota = lax.broadcasted_iota(jnp.int32, (TOPK_BLK, K), 1)

    def it(t, carry):
        dist, idx_acc, d_acc = carry
        mn = jnp.min(dist, axis=1, keepdims=True)
        am = jnp.min(jnp.where(dist == mn, col_iota, N), axis=1, keepdims=True)
        dist = jnp.where(col_iota == am, jnp.inf, dist)
        sel = sel_iota == t
        idx_acc = jnp.where(sel, am, idx_acc)
        d_acc = jnp.where(sel, mn, d_acc)
        return dist, idx_acc, d_acc

    _, idx_acc, d_acc = lax.fori_loop(
        0, K, it,
        (dist,
         jnp.zeros((TOPK_BLK, K), jnp.int32),
         jnp.zeros((TOPK_BLK, K), jnp.float32)),
    )
    idx_ref[...] = idx_acc
    d_ref[...] = d_acc


def _topk(coords, coords_t):
    return pl.pallas_call(
        _topk_body,
        grid=(N // TOPK_BLK,),
        in_specs=[
            pl.BlockSpec((TOPK_BLK, 3), lambda i: (i, 0)),
            pl.BlockSpec((3, N), lambda i: (0, 0)),
        ],
        out_specs=[
            pl.BlockSpec((TOPK_BLK, K), lambda i: (i, 0)),
            pl.BlockSpec((TOPK_BLK, K), lambda i: (i, 0)),
        ],
        out_shape=[
            jax.ShapeDtypeStruct((N, K), jnp.int32),
            jax.ShapeDtypeStruct((N, K), jnp.float32),
        ],
    )(coords, coords_t)


def _gather_body(feats_hbm, idx_hbm, out_hbm, idx_v, rows_v, sem):
    wid = lax.axis_index("s") * 2 + lax.axis_index("c")
    base = wid * ROWS_PER_W
    pltpu.sync_copy(idx_hbm.at[pl.ds(base, ROWS_PER_W)], idx_v)

    def body(j, _):
        pltpu.async_copy(
            feats_hbm.at[idx_v.at[pl.ds(j * GCH, GCH)]], rows_v, sem
        ).wait()
        pltpu.sync_copy(rows_v, out_hbm.at[pl.ds(base + j * GCH, GCH)])
        return 0

    lax.fori_loop(0, NCH, body, 0)


@functools.cache
def _make_sc_gather():
    return pl.kernel(
        _gather_body,
        out_type=jax.ShapeDtypeStruct((N * K, DIM), jnp.float32),
        mesh=plsc.VectorSubcoreMesh(core_axis_name="c", subcore_axis_name="s"),
        scratch_types=[
            pltpu.VMEM((ROWS_PER_W,), jnp.int32),
            pltpu.VMEM((GCH, DIM), jnp.float32),
            pltpu.SemaphoreType.DMA,
        ],
    )


def _sc_gather(feats, idx_flat):
    return _make_sc_gather()(feats, idx_flat)


def _layer_body(feats_ref, fj_ref, d_ref, w1_ref, w1c_ref, b1_ref, w2_ref,
                b2_ref, wg_ref, bg_ref, lng_ref, lnb_ref, wn1_ref, bn1_ref,
                wn2_ref, bn2_ref, out_ref, p_scr, acc_scr):
    t = pl.program_id(0)

    @pl.when(t == 0)
    def _():
        p_scr[...] = (
            jnp.dot(feats_ref[...].astype(jnp.bfloat16), w1_ref[0:DIM, :],
                    preferred_element_type=jnp.float32)
            + b1_ref[...]
        )
        acc_scr[...] = jnp.zeros((N, MDIM), jnp.float32)

    fj = fj_ref[0].astype(jnp.bfloat16)
    qk = jnp.dot(fj, w1_ref[DIM : 2 * DIM, :],
                 preferred_element_type=jnp.float32)
    onehot = (lax.broadcasted_iota(jnp.int32, (K, 1), 0) == t).astype(jnp.float32)
    dcol = jnp.dot(d_ref[...], onehot, preferred_element_type=jnp.float32)
    pre = p_scr[...] + qk + dcol * w1c_ref[...]
    h = (pre * jax.nn.sigmoid(pre)).astype(jnp.bfloat16)
    mk = _silu(jnp.dot(h, w2_ref[...], preferred_element_type=jnp.float32)
               + b2_ref[...])
    gate = jax.nn.sigmoid(
        jnp.dot(mk, wg_ref[...], preferred_element_type=jnp.float32)
        + bg_ref[...]
    )
    acc_scr[...] += mk * gate

    @pl.when(t == K - 1)
    def _():
        feats = feats_ref[...]
        m_i = acc_scr[...]
        mu = jnp.mean(feats, axis=-1, keepdims=True)
        var = jnp.mean((feats - mu) ** 2, axis=-1, keepdims=True)
        normed = (feats - mu) / jnp.sqrt(var + 1e-5) * lng_ref[...] + lnb_ref[...]
        nh = _silu(
            jnp.dot(normed.astype(jnp.bfloat16), wn1_ref[0:DIM, :],
                    preferred_element_type=jnp.float32)
            + jnp.dot(m_i.astype(jnp.bfloat16), wn1_ref[DIM : DIM + MDIM, :],
                      preferred_element_type=jnp.float32)
            + bn1_ref[...]
        ).astype(jnp.bfloat16)
        out_ref[...] = (
            jnp.dot(nh, wn2_ref[...], preferred_element_type=jnp.float32)
            + bn2_ref[...]
            + feats
        )


def _layer(feats, fj, d, w1, b1, w2, b2, wg, bg, lng, lnb, wn1, bn1, wn2, bn2):
    whole = lambda shape: pl.BlockSpec(shape, lambda t: tuple(0 for _ in shape))
    return pl.pallas_call(
        _layer_body,
        grid=(K,),
        in_specs=[
            whole((N, DIM)),
            pl.BlockSpec((1, N, DIM), lambda t: (t, 0, 0)),
            whole((N, K)),
            whole((2 * DIM, HID)),
            whole((1, HID)),
            whole((1, HID)),
            whole((HID, MDIM)),
            whole((1, MDIM)),
            whole((MDIM, 1)),
            whole((1, 1)),
            whole((1, DIM)),
            whole((1, DIM)),
            whole((DIM + MDIM, 2 * DIM)),
            whole((1, 2 * DIM)),
            whole((2 * DIM, DIM)),
            whole((1, DIM)),
        ],
        out_specs=whole((N, DIM)),
        out_shape=jax.ShapeDtypeStruct((N, DIM), jnp.float32),
        scratch_shapes=[
            pltpu.VMEM((N, HID), jnp.float32),
            pltpu.VMEM((N, MDIM), jnp.float32),
        ],
    )(feats, fj.reshape(K, N, DIM), d,
      w1[0 : 2 * DIM].astype(jnp.bfloat16), w1[2 * DIM :], b1.reshape(1, HID),
      w2.astype(jnp.bfloat16), b2.reshape(1, MDIM), wg, bg.reshape(1, 1),
      lng.reshape(1, DIM), lnb.reshape(1, DIM),
      wn1.astype(jnp.bfloat16), bn1.reshape(1, 2 * DIM),
      wn2.astype(jnp.bfloat16), bn2.reshape(1, DIM))


def kernel(feats, coords, W1, b1, W2, b2, Wg, bg, ln_g, ln_b, Wn1, bn1, Wn2, bn2):
    f = feats[0]
    c = coords[0]
    idx, d = _topk(c, c.T)
    idx_flat = idx.T.reshape(-1)
    for l in range(3):
        fj = _sc_gather(f, idx_flat)
        f = _layer(f, fj, d, W1[l], b1[l], W2[l], b2[l], Wg[l], bg[l],
                   ln_g[l], ln_b[l], Wn1[l], bn1[l], Wn2[l], bn2[l])
    return f[None]


# speedup vs baseline: 1.1208x; 1.1208x over previous
import functools

import jax
import jax.numpy as jnp
from jax import lax
from jax.experimental import pallas as pl
from jax.experimental.pallas import tpu as pltpu
from jax.experimental.pallas import tpu_sc as plsc

N = 2048
DIM = 128
K = 30
EI = 2 * DIM + 1
HID = 2 * EI
MDIM = 16
TOPK_BLK = 256
N_WORKERS = 32
ROWS_PER_W = (N * K) // N_WORKERS
GCH = 128
NCH = ROWS_PER_W // GCH


def _silu(x):
    return x * jax.nn.sigmoid(x)


def _topk_body(coords_ref, coords_t_ref, idx_ref, d_ref):
    dist = jnp.zeros((TOPK_BLK, N), jnp.float32)
    for c in range(3):
        col = coords_ref[:, c : c + 1]
        row = coords_t_ref[c : c + 1, :]
        diff = col - row
        dist = dist + diff * diff
    col_iota = lax.broadcasted_iota(jnp.int32, (TOPK_BLK, N), 1)
    sel_iota = lax.broadcasted_iota(jnp.int32, (TOPK_BLK, K), 1)

    def it(t, carry):
        dist, idx_acc, d_acc = carry
        mn = jnp.min(dist, axis=1, keepdims=True)
        am = jnp.min(jnp.where(dist == mn, col_iota, N), axis=1, keepdims=True)
        dist = jnp.where(col_iota == am, jnp.inf, dist)
        sel = sel_iota == t
        idx_acc = jnp.where(sel, am, idx_acc)
        d_acc = jnp.where(sel, mn, d_acc)
        return dist, idx_acc, d_acc

    _, idx_acc, d_acc = lax.fori_loop(
        0, K, it,
        (dist,
         jnp.zeros((TOPK_BLK, K), jnp.int32),
         jnp.zeros((TOPK_BLK, K), jnp.float32)),
    )
    idx_ref[...] = idx_acc
    d_ref[...] = d_acc


def _topk(coords, coords_t):
    return pl.pallas_call(
        _topk_body,
        grid=(N // TOPK_BLK,),
        in_specs=[
            pl.BlockSpec((TOPK_BLK, 3), lambda i: (i, 0)),
            pl.BlockSpec((3, N), lambda i: (0, 0)),
        ],
        out_specs=[
            pl.BlockSpec((TOPK_BLK, K), lambda i: (i, 0)),
            pl.BlockSpec((TOPK_BLK, K), lambda i: (i, 0)),
        ],
        out_shape=[
            jax.ShapeDtypeStruct((N, K), jnp.int32),
            jax.ShapeDtypeStruct((N, K), jnp.float32),
        ],
    )(coords, coords_t)


def _gather_body(feats_hbm, idx_hbm, out_hbm, idx_v, rows_v, sem):
    wid = lax.axis_index("s") * 2 + lax.axis_index("c")
    base = wid * ROWS_PER_W
    pltpu.sync_copy(idx_hbm.at[pl.ds(base, ROWS_PER_W)], idx_v)

    def body(j, _):
        pltpu.async_copy(
            feats_hbm.at[idx_v.at[pl.ds(j * GCH, GCH)]], rows_v, sem
        ).wait()
        pltpu.sync_copy(rows_v, out_hbm.at[pl.ds(base + j * GCH, GCH)])
        return 0

    lax.fori_loop(0, NCH, body, 0)


@functools.cache
def _make_sc_gather():
    return pl.kernel(
        _gather_body,
        out_type=jax.ShapeDtypeStruct((N * K, DIM), jnp.float32),
        mesh=plsc.VectorSubcoreMesh(core_axis_name="c", subcore_axis_name="s"),
        scratch_types=[
            pltpu.VMEM((ROWS_PER_W,), jnp.int32),
            pltpu.VMEM((GCH, DIM), jnp.float32),
            pltpu.SemaphoreType.DMA,
        ],
    )


def _sc_gather(feats, idx_flat):
    return _make_sc_gather()(feats, idx_flat)


S = 6
NSTEP = K // S


def _fsilu(x):
    return x / (1.0 + jnp.exp(-x))


def _layer_body(feats_ref, fj_ref, d_ref, w1_ref, b1_ref, w2_ref, b26_ref,
                wg6_ref, exp6_ref, bg_ref, lng_ref, lnb_ref, wn1_ref, bn1_ref,
                wn2_ref, bn2_ref, out_ref, p_scr, acc_scr):
    t = pl.program_id(0)

    @pl.when(t == 0)
    def _():
        p_scr[...] = (
            jnp.dot(feats_ref[...], w1_ref[0:DIM, :],
                    preferred_element_type=jnp.float32)
            + b1_ref[...]
        )
        acc_scr[...] = jnp.zeros((N, S * MDIM), jnp.float32)

    rowi = lax.broadcasted_iota(jnp.int32, (K, S), 0)
    colj = lax.broadcasted_iota(jnp.int32, (K, S), 1)
    onehot6 = (rowi == S * t + colj).astype(jnp.float32)
    d6 = jnp.dot(d_ref[...], onehot6, preferred_element_type=jnp.float32)

    p = p_scr[...]
    w1b = w1_ref[DIM : 2 * DIM, :]
    w1c = w1_ref[2 * DIM : 2 * DIM + 1, :]
    mks = []
    for s in range(S):
        qk = jnp.dot(fj_ref[s], w1b, preferred_element_type=jnp.float32)
        pre = p + qk + d6[:, s : s + 1] * w1c
        h = _fsilu(pre)
        mks.append(jnp.dot(h, w2_ref[...], preferred_element_type=jnp.float32))
    mk_cat = _fsilu(jnp.concatenate(mks, axis=1) + b26_ref[...])
    gpre = jnp.dot(mk_cat, wg6_ref[...], preferred_element_type=jnp.float32) \
        + bg_ref[...]
    g = 1.0 / (1.0 + jnp.exp(-gpre))
    gexp = jnp.dot(g, exp6_ref[...], preferred_element_type=jnp.float32)
    acc_scr[...] += mk_cat * gexp

    @pl.when(t == NSTEP - 1)
    def _():
        feats = feats_ref[...]
        acc = acc_scr[...]
        m_i = acc[:, 0:MDIM]
        for s in range(1, S):
            m_i = m_i + acc[:, s * MDIM : (s + 1) * MDIM]
        mu = jnp.mean(feats, axis=-1, keepdims=True)
        var = jnp.mean((feats - mu) ** 2, axis=-1, keepdims=True)
        normed = (feats - mu) / jnp.sqrt(var + 1e-5) * lng_ref[...] + lnb_ref[...]
        nh = _fsilu(
            jnp.dot(normed, wn1_ref[0:DIM, :], preferred_element_type=jnp.float32)
            + jnp.dot(m_i, wn1_ref[DIM : DIM + MDIM, :],
                      preferred_element_type=jnp.float32)
            + bn1_ref[...]
        )
        out_ref[...] = (
            jnp.dot(nh, wn2_ref[...], preferred_element_type=jnp.float32)
            + bn2_ref[...]
            + feats
        )


def _layer(feats, fj, d, w1, b1, w2, b2, wg, bg, lng, lnb, wn1, bn1, wn2, bn2):
    whole = lambda shape: pl.BlockSpec(shape, lambda t: tuple(0 for _ in shape))
    b26 = jnp.tile(b2.reshape(1, MDIM), (1, S))
    wg6 = jnp.kron(jnp.eye(S, dtype=jnp.float32), wg)
    exp6 = jnp.kron(jnp.eye(S, dtype=jnp.float32),
                    jnp.ones((1, MDIM), jnp.float32))
    return pl.pallas_call(
        _layer_body,
        grid=(NSTEP,),
        in_specs=[
            whole((N, DIM)),
            pl.BlockSpec((S, N, DIM), lambda t: (t, 0, 0)),
            whole((N, K)),
            whole((EI, HID)),
            whole((1, HID)),
            whole((HID, MDIM)),
            whole((1, S * MDIM)),
            whole((S * MDIM, S)),
            whole((S, S * MDIM)),
            whole((1, 1)),
            whole((1, DIM)),
            whole((1, DIM)),
            whole((DIM + MDIM, 2 * DIM)),
            whole((1, 2 * DIM)),
            whole((2 * DIM, DIM)),
            whole((1, DIM)),
        ],
        out_specs=whole((N, DIM)),
        out_shape=jax.ShapeDtypeStruct((N, DIM), jnp.float32),
        scratch_shapes=[
            pltpu.VMEM((N, HID), jnp.float32),
            pltpu.VMEM((N, S * MDIM), jnp.float32),
        ],
    )(feats, fj.reshape(K, N, DIM), d, w1, b1.reshape(1, HID),
      w2, b26, wg6, exp6, bg.reshape(1, 1),
      lng.reshape(1, DIM), lnb.reshape(1, DIM),
      wn1, bn1.reshape(1, 2 * DIM), wn2, bn2.reshape(1, DIM))


def kernel(feats, coords, W1, b1, W2, b2, Wg, bg, ln_g, ln_b, Wn1, bn1, Wn2, bn2):
    f = feats[0]
    c = coords[0]
    idx, d = _topk(c, c.T)
    idx_flat = idx.T.reshape(-1)
    for l in range(3):
        fj = _sc_gather(f, idx_flat)
        f = _layer(f, fj, d, W1[l], b1[l], W2[l], b2[l], Wg[l], bg[l],
                   ln_g[l], ln_b[l], Wn1[l], bn1[l], Wn2[l], bn2[l])
    return f[None]


# speedup vs baseline: 1.2748x; 1.1374x over previous
import functools

import jax
import jax.numpy as jnp
from jax import lax
from jax.experimental import pallas as pl
from jax.experimental.pallas import tpu as pltpu
from jax.experimental.pallas import tpu_sc as plsc

N = 2048
DIM = 128
K = 30
EI = 2 * DIM + 1
HID = 2 * EI
MDIM = 16
TOPK_BLK = 256
N_WORKERS = 32
ROWS_PER_W = (N * K) // N_WORKERS
GCH = 128
NCH = ROWS_PER_W // GCH


def _silu(x):
    return x * jax.nn.sigmoid(x)


def _topk_body(coords_ref, coords_t_ref, idx_ref, d_ref):
    dist = jnp.zeros((TOPK_BLK, N), jnp.float32)
    for c in range(3):
        col = coords_ref[:, c : c + 1]
        row = coords_t_ref[c : c + 1, :]
        diff = col - row
        dist = dist + diff * diff
    col_iota = lax.broadcasted_iota(jnp.int32, (TOPK_BLK, N), 1)
    sel_iota = lax.broadcasted_iota(jnp.int32, (TOPK_BLK, K), 1)

    def it(t, carry):
        dist, idx_acc, d_acc = carry
        mn = jnp.min(dist, axis=1, keepdims=True)
        am = jnp.min(jnp.where(dist == mn, col_iota, N), axis=1, keepdims=True)
        dist = jnp.where(col_iota == am, jnp.inf, dist)
        sel = sel_iota == t
        idx_acc = jnp.where(sel, am, idx_acc)
        d_acc = jnp.where(sel, mn, d_acc)
        return dist, idx_acc, d_acc

    _, idx_acc, d_acc = lax.fori_loop(
        0, K, it,
        (dist,
         jnp.zeros((TOPK_BLK, K), jnp.int32),
         jnp.zeros((TOPK_BLK, K), jnp.float32)),
        unroll=5,
    )
    idx_ref[...] = idx_acc
    d_ref[...] = d_acc


def _topk(coords, coords_t):
    return pl.pallas_call(
        _topk_body,
        grid=(N // TOPK_BLK,),
        in_specs=[
            pl.BlockSpec((TOPK_BLK, 3), lambda i: (i, 0)),
            pl.BlockSpec((3, N), lambda i: (0, 0)),
        ],
        out_specs=[
            pl.BlockSpec((TOPK_BLK, K), lambda i: (i, 0)),
            pl.BlockSpec((TOPK_BLK, K), lambda i: (i, 0)),
        ],
        out_shape=[
            jax.ShapeDtypeStruct((N, K), jnp.int32),
            jax.ShapeDtypeStruct((N, K), jnp.float32),
        ],
    )(coords, coords_t)


def _gather_body(feats_hbm, idx_hbm, out_hbm, idx_v, rows_v, sem):
    wid = lax.axis_index("s") * 2 + lax.axis_index("c")
    base = wid * ROWS_PER_W
    pltpu.sync_copy(idx_hbm.at[pl.ds(base, ROWS_PER_W)], idx_v)

    def body(j, _):
        pltpu.async_copy(
            feats_hbm.at[idx_v.at[pl.ds(j * GCH, GCH)]], rows_v, sem
        ).wait()
        pltpu.sync_copy(rows_v, out_hbm.at[pl.ds(base + j * GCH, GCH)])
        return 0

    lax.fori_loop(0, NCH, body, 0)


@functools.cache
def _make_sc_gather():
    return pl.kernel(
        _gather_body,
        out_type=jax.ShapeDtypeStruct((N * K, DIM), jnp.float32),
        mesh=plsc.VectorSubcoreMesh(core_axis_name="c", subcore_axis_name="s"),
        scratch_types=[
            pltpu.VMEM((ROWS_PER_W,), jnp.int32),
            pltpu.VMEM((GCH, DIM), jnp.float32),
            pltpu.SemaphoreType.DMA,
        ],
    )


def _sc_gather(feats, idx_flat):
    return _make_sc_gather()(feats, idx_flat)


S = 6
NSTEP = K // S


def _fsilu(x):
    return x / (1.0 + jnp.exp(-x))


def _layer_body(feats_ref, fj_ref, d_ref, w1_ref, b1_ref, w2_ref, b26_ref,
                wg6_ref, exp6_ref, bg_ref, lng_ref, lnb_ref, wn1_ref, bn1_ref,
                wn2_ref, bn2_ref, out_ref, p_scr, acc_scr):
    t = pl.program_id(0)

    @pl.when(t == 0)
    def _():
        p_scr[...] = (
            jnp.dot(feats_ref[...], w1_ref[0:DIM, :],
                    preferred_element_type=jnp.float32)
            + b1_ref[...]
        )
        acc_scr[...] = jnp.zeros((N, S * MDIM), jnp.float32)

    rowi = lax.broadcasted_iota(jnp.int32, (K, S), 0)
    colj = lax.broadcasted_iota(jnp.int32, (K, S), 1)
    onehot6 = (rowi == S * t + colj).astype(jnp.float32)
    d6 = jnp.dot(d_ref[...], onehot6, preferred_element_type=jnp.float32)

    p = p_scr[...]
    w1b = w1_ref[DIM : 2 * DIM, :]
    w1c = w1_ref[2 * DIM : 2 * DIM + 1, :]
    mks = []
    for s in range(S):
        qk = jnp.dot(fj_ref[s], w1b, preferred_element_type=jnp.float32)
        pre = p + qk + d6[:, s : s + 1] * w1c
        h = _fsilu(pre)
        mks.append(jnp.dot(h, w2_ref[...], preferred_element_type=jnp.float32))
    mk_cat = _fsilu(jnp.concatenate(mks, axis=1) + b26_ref[...])
    gpre = jnp.dot(mk_cat, wg6_ref[...], preferred_element_type=jnp.float32) \
        + bg_ref[...]
    g = 1.0 / (1.0 + jnp.exp(-gpre))
    gexp = jnp.dot(g, exp6_ref[...], preferred_element_type=jnp.float32)
    acc_scr[...] += mk_cat * gexp

    @pl.when(t == NSTEP - 1)
    def _():
        feats = feats_ref[...]
        acc = acc_scr[...]
        m_i = acc[:, 0:MDIM]
        for s in range(1, S):
            m_i = m_i + acc[:, s * MDIM : (s + 1) * MDIM]
        mu = jnp.mean(feats, axis=-1, keepdims=True)
        var = jnp.mean((feats - mu) ** 2, axis=-1, keepdims=True)
        normed = (feats - mu) / jnp.sqrt(var + 1e-5) * lng_ref[...] + lnb_ref[...]
        nh = _fsilu(
            jnp.dot(normed, wn1_ref[0:DIM, :], preferred_element_type=jnp.float32)
            + jnp.dot(m_i, wn1_ref[DIM : DIM + MDIM, :],
                      preferred_element_type=jnp.float32)
            + bn1_ref[...]
        )
        out_ref[...] = (
            jnp.dot(nh, wn2_ref[...], preferred_element_type=jnp.float32)
            + bn2_ref[...]
            + feats
        )


def _layer(feats, fj, d, w1, b1, w2, b2, wg, bg, lng, lnb, wn1, bn1, wn2, bn2):
    whole = lambda shape: pl.BlockSpec(shape, lambda t: tuple(0 for _ in shape))
    b26 = jnp.tile(b2.reshape(1, MDIM), (1, S))
    wg6 = jnp.kron(jnp.eye(S, dtype=jnp.float32), wg)
    exp6 = jnp.kron(jnp.eye(S, dtype=jnp.float32),
                    jnp.ones((1, MDIM), jnp.float32))
    return pl.pallas_call(
        _layer_body,
        grid=(NSTEP,),
        in_specs=[
            whole((N, DIM)),
            pl.BlockSpec((S, N, DIM), lambda t: (t, 0, 0)),
            whole((N, K)),
            whole((EI, HID)),
            whole((1, HID)),
            whole((HID, MDIM)),
            whole((1, S * MDIM)),
            whole((S * MDIM, S)),
            whole((S, S * MDIM)),
            whole((1, 1)),
            whole((1, DIM)),
            whole((1, DIM)),
            whole((DIM + MDIM, 2 * DIM)),
            whole((1, 2 * DIM)),
            whole((2 * DIM, DIM)),
            whole((1, DIM)),
        ],
        out_specs=whole((N, DIM)),
        out_shape=jax.ShapeDtypeStruct((N, DIM), jnp.float32),
        scratch_shapes=[
            pltpu.VMEM((N, HID), jnp.float32),
            pltpu.VMEM((N, S * MDIM), jnp.float32),
        ],
    )(feats, fj.reshape(K, N, DIM), d, w1, b1.reshape(1, HID),
      w2, b26, wg6, exp6, bg.reshape(1, 1),
      lng.reshape(1, DIM), lnb.reshape(1, DIM),
      wn1, bn1.reshape(1, 2 * DIM), wn2, bn2.reshape(1, DIM))


def kernel(feats, coords, W1, b1, W2, b2, Wg, bg, ln_g, ln_b, Wn1, bn1, Wn2, bn2):
    f = feats[0]
    c = coords[0]
    idx, d = _topk(c, c.T)
    idx_flat = idx.T.reshape(-1)
    for l in range(3):
        fj = _sc_gather(f, idx_flat)
        f = _layer(f, fj, d, W1[l], b1[l], W2[l], b2[l], Wg[l], bg[l],
                   ln_g[l], ln_b[l], Wn1[l], bn1[l], Wn2[l], bn2[l])
    return f[None]


# speedup vs baseline: 1.3049x; 1.0236x over previous
import functools

import jax
import jax.numpy as jnp
from jax import lax
from jax.experimental import pallas as pl
from jax.experimental.pallas import tpu as pltpu
from jax.experimental.pallas import tpu_sc as plsc

N = 2048
DIM = 128
K = 30
EI = 2 * DIM + 1
HID = 2 * EI
MDIM = 16
TOPK_BLK = 256
N_WORKERS = 32
ROWS_PER_W = (N * K) // N_WORKERS
GCH = 128
NCH = ROWS_PER_W // GCH


def _silu(x):
    return x * jax.nn.sigmoid(x)


def _topk_body(coords_ref, coords_t_ref, idx_ref, d_ref):
    dist = jnp.zeros((TOPK_BLK, N), jnp.float32)
    for c in range(3):
        col = coords_ref[:, c : c + 1]
        row = coords_t_ref[c : c + 1, :]
        diff = col - row
        dist = dist + diff * diff
    col_iota = lax.broadcasted_iota(jnp.int32, (TOPK_BLK, N), 1)
    sel_iota = lax.broadcasted_iota(jnp.int32, (TOPK_BLK, K), 1)

    def it(t, carry):
        dist, idx_acc, d_acc = carry
        mn = jnp.min(dist, axis=1, keepdims=True)
        am = jnp.min(jnp.where(dist == mn, col_iota, N), axis=1, keepdims=True)
        dist = jnp.where(col_iota == am, jnp.inf, dist)
        sel = sel_iota == t
        idx_acc = jnp.where(sel, am, idx_acc)
        d_acc = jnp.where(sel, mn, d_acc)
        return dist, idx_acc, d_acc

    _, idx_acc, d_acc = lax.fori_loop(
        0, K, it,
        (dist,
         jnp.zeros((TOPK_BLK, K), jnp.int32),
         jnp.zeros((TOPK_BLK, K), jnp.float32)),
        unroll=10,
    )
    idx_ref[...] = idx_acc
    d_ref[...] = d_acc


def _topk(coords, coords_t):
    return pl.pallas_call(
        _topk_body,
        grid=(N // TOPK_BLK,),
        in_specs=[
            pl.BlockSpec((TOPK_BLK, 3), lambda i: (i, 0)),
            pl.BlockSpec((3, N), lambda i: (0, 0)),
        ],
        out_specs=[
            pl.BlockSpec((TOPK_BLK, K), lambda i: (i, 0)),
            pl.BlockSpec((TOPK_BLK, K), lambda i: (i, 0)),
        ],
        out_shape=[
            jax.ShapeDtypeStruct((N, K), jnp.int32),
            jax.ShapeDtypeStruct((N, K), jnp.float32),
        ],
    )(coords, coords_t)


def _gather_body(feats_hbm, idx_hbm, out_hbm, idx_v, rows_v, sem):
    wid = lax.axis_index("s") * 2 + lax.axis_index("c")
    base = wid * ROWS_PER_W
    pltpu.sync_copy(idx_hbm.at[pl.ds(base, ROWS_PER_W)], idx_v)

    def body(j, _):
        pltpu.async_copy(
            feats_hbm.at[idx_v.at[pl.ds(j * GCH, GCH)]], rows_v, sem
        ).wait()
        pltpu.sync_copy(rows_v, out_hbm.at[pl.ds(base + j * GCH, GCH)])
        return 0

    lax.fori_loop(0, NCH, body, 0)


@functools.cache
def _make_sc_gather():
    return pl.kernel(
        _gather_body,
        out_type=jax.ShapeDtypeStruct((N * K, DIM), jnp.float32),
        mesh=plsc.VectorSubcoreMesh(core_axis_name="c", subcore_axis_name="s"),
        scratch_types=[
            pltpu.VMEM((ROWS_PER_W,), jnp.int32),
            pltpu.VMEM((GCH, DIM), jnp.float32),
            pltpu.SemaphoreType.DMA,
        ],
    )


def _sc_gather(feats, idx_flat):
    return _make_sc_gather()(feats, idx_flat)


S = 6
NSTEP = K // S


def _fsilu(x):
    return x / (1.0 + jnp.exp(-x))


def _layer_body(feats_ref, fj_ref, d_ref, w1_ref, b1_ref, w2_ref, b26_ref,
                wg6_ref, exp6_ref, bg_ref, lng_ref, lnb_ref, wn1_ref, bn1_ref,
                wn2_ref, bn2_ref, out_ref, p_scr, acc_scr):
    t = pl.program_id(0)

    @pl.when(t == 0)
    def _():
        p_scr[...] = (
            jnp.dot(feats_ref[...], w1_ref[0:DIM, :],
                    preferred_element_type=jnp.float32)
            + b1_ref[...]
        )
        acc_scr[...] = jnp.zeros((N, S * MDIM), jnp.float32)

    rowi = lax.broadcasted_iota(jnp.int32, (K, S), 0)
    colj = lax.broadcasted_iota(jnp.int32, (K, S), 1)
    onehot6 = (rowi == S * t + colj).astype(jnp.float32)
    d6 = jnp.dot(d_ref[...], onehot6, preferred_element_type=jnp.float32)

    p = p_scr[...]
    w1b = w1_ref[DIM : 2 * DIM, :]
    w1c = w1_ref[2 * DIM : 2 * DIM + 1, :]
    mks = []
    for s in range(S):
        qk = jnp.dot(fj_ref[s], w1b, preferred_element_type=jnp.float32)
        pre = p + qk + d6[:, s : s + 1] * w1c
        h = _fsilu(pre)
        mks.append(jnp.dot(h, w2_ref[...], preferred_element_type=jnp.float32))
    mk_cat = _fsilu(jnp.concatenate(mks, axis=1) + b26_ref[...])
    gpre = jnp.dot(mk_cat, wg6_ref[...], preferred_element_type=jnp.float32) \
        + bg_ref[...]
    g = 1.0 / (1.0 + jnp.exp(-gpre))
    gexp = jnp.dot(g, exp6_ref[...], preferred_element_type=jnp.float32)
    acc_scr[...] += mk_cat * gexp

    @pl.when(t == NSTEP - 1)
    def _():
        feats = feats_ref[...]
        acc = acc_scr[...]
        m_i = acc[:, 0:MDIM]
        for s in range(1, S):
            m_i = m_i + acc[:, s * MDIM : (s + 1) * MDIM]
        mu = jnp.mean(feats, axis=-1, keepdims=True)
        var = jnp.mean((feats - mu) ** 2, axis=-1, keepdims=True)
        normed = (feats - mu) / jnp.sqrt(var + 1e-5) * lng_ref[...] + lnb_ref[...]
        nh = _fsilu(
            jnp.dot(normed, wn1_ref[0:DIM, :], preferred_element_type=jnp.float32)
            + jnp.dot(m_i, wn1_ref[DIM : DIM + MDIM, :],
                      preferred_element_type=jnp.float32)
            + bn1_ref[...]
        )
        out_ref[...] = (
            jnp.dot(nh, wn2_ref[...], preferred_element_type=jnp.float32)
            + bn2_ref[...]
            + feats
        )


def _layer(feats, fj, d, w1, b1, w2, b2, wg, bg, lng, lnb, wn1, bn1, wn2, bn2):
    whole = lambda shape: pl.BlockSpec(shape, lambda t: tuple(0 for _ in shape))
    b26 = jnp.tile(b2.reshape(1, MDIM), (1, S))
    wg6 = jnp.kron(jnp.eye(S, dtype=jnp.float32), wg)
    exp6 = jnp.kron(jnp.eye(S, dtype=jnp.float32),
                    jnp.ones((1, MDIM), jnp.float32))
    return pl.pallas_call(
        _layer_body,
        grid=(NSTEP,),
        in_specs=[
            whole((N, DIM)),
            pl.BlockSpec((S, N, DIM), lambda t: (t, 0, 0)),
            whole((N, K)),
            whole((EI, HID)),
            whole((1, HID)),
            whole((HID, MDIM)),
            whole((1, S * MDIM)),
            whole((S * MDIM, S)),
            whole((S, S * MDIM)),
            whole((1, 1)),
            whole((1, DIM)),
            whole((1, DIM)),
            whole((DIM + MDIM, 2 * DIM)),
            whole((1, 2 * DIM)),
            whole((2 * DIM, DIM)),
            whole((1, DIM)),
        ],
        out_specs=whole((N, DIM)),
        out_shape=jax.ShapeDtypeStruct((N, DIM), jnp.float32),
        scratch_shapes=[
            pltpu.VMEM((N, HID), jnp.float32),
            pltpu.VMEM((N, S * MDIM), jnp.float32),
        ],
    )(feats, fj.reshape(K, N, DIM), d, w1, b1.reshape(1, HID),
      w2, b26, wg6, exp6, bg.reshape(1, 1),
      lng.reshape(1, DIM), lnb.reshape(1, DIM),
      wn1, bn1.reshape(1, 2 * DIM), wn2, bn2.reshape(1, DIM))


def kernel(feats, coords, W1, b1, W2, b2, Wg, bg, ln_g, ln_b, Wn1, bn1, Wn2, bn2):
    f = feats[0]
    c = coords[0]
    idx, d = _topk(c, c.T)
    idx_flat = idx.T.reshape(-1)
    for l in range(3):
        fj = _sc_gather(f, idx_flat)
        f = _layer(f, fj, d, W1[l], b1[l], W2[l], b2[l], Wg[l], bg[l],
                   ln_g[l], ln_b[l], Wn1[l], bn1[l], Wn2[l], bn2[l])
    return f[None]


# speedup vs baseline: 1.3193x; 1.0110x over previous
import functools

import jax
import jax.numpy as jnp
from jax import lax
from jax.experimental import pallas as pl
from jax.experimental.pallas import tpu as pltpu
from jax.experimental.pallas import tpu_sc as plsc

N = 2048
DIM = 128
K = 30
EI = 2 * DIM + 1
HID = 2 * EI
MDIM = 16
TOPK_BLK = 256
N_WORKERS = 32
ROWS_PER_W = (N * K) // N_WORKERS
GCH = 128
NCH = ROWS_PER_W // GCH


def _silu(x):
    return x * jax.nn.sigmoid(x)


def _topk_body(coords_ref, coords_t_ref, idx_ref, d_ref):
    dist = jnp.zeros((TOPK_BLK, N), jnp.float32)
    for c in range(3):
        col = coords_ref[:, c : c + 1]
        row = coords_t_ref[c : c + 1, :]
        diff = col - row
        dist = dist + diff * diff
    col_iota = lax.broadcasted_iota(jnp.int32, (TOPK_BLK, N), 1)
    sel_iota = lax.broadcasted_iota(jnp.int32, (TOPK_BLK, K), 1)

    def it(t, carry):
        dist, idx_acc, d_acc = carry
        mn = jnp.min(dist, axis=1, keepdims=True)
        am = jnp.min(jnp.where(dist == mn, col_iota, N), axis=1, keepdims=True)
        dist = jnp.where(col_iota == am, jnp.inf, dist)
        sel = sel_iota == t
        idx_acc = jnp.where(sel, am, idx_acc)
        d_acc = jnp.where(sel, mn, d_acc)
        return dist, idx_acc, d_acc

    _, idx_acc, d_acc = lax.fori_loop(
        0, K, it,
        (dist,
         jnp.zeros((TOPK_BLK, K), jnp.int32),
         jnp.zeros((TOPK_BLK, K), jnp.float32)),
        unroll=15,
    )
    idx_ref[...] = idx_acc
    d_ref[...] = d_acc


def _topk(coords, coords_t):
    return pl.pallas_call(
        _topk_body,
        grid=(N // TOPK_BLK,),
        in_specs=[
            pl.BlockSpec((TOPK_BLK, 3), lambda i: (i, 0)),
            pl.BlockSpec((3, N), lambda i: (0, 0)),
        ],
        out_specs=[
            pl.BlockSpec((TOPK_BLK, K), lambda i: (i, 0)),
            pl.BlockSpec((TOPK_BLK, K), lambda i: (i, 0)),
        ],
        out_shape=[
            jax.ShapeDtypeStruct((N, K), jnp.int32),
            jax.ShapeDtypeStruct((N, K), jnp.float32),
        ],
    )(coords, coords_t)


def _gather_body(feats_hbm, idx_hbm, out_hbm, idx_v, rows_v, sem):
    wid = lax.axis_index("s") * 2 + lax.axis_index("c")
    base = wid * ROWS_PER_W
    pltpu.sync_copy(idx_hbm.at[pl.ds(base, ROWS_PER_W)], idx_v)

    def body(j, _):
        pltpu.async_copy(
            feats_hbm.at[idx_v.at[pl.ds(j * GCH, GCH)]], rows_v, sem
        ).wait()
        pltpu.sync_copy(rows_v, out_hbm.at[pl.ds(base + j * GCH, GCH)])
        return 0

    lax.fori_loop(0, NCH, body, 0)


@functools.cache
def _make_sc_gather():
    return pl.kernel(
        _gather_body,
        out_type=jax.ShapeDtypeStruct((N * K, DIM), jnp.float32),
        mesh=plsc.VectorSubcoreMesh(core_axis_name="c", subcore_axis_name="s"),
        scratch_types=[
            pltpu.VMEM((ROWS_PER_W,), jnp.int32),
            pltpu.VMEM((GCH, DIM), jnp.float32),
            pltpu.SemaphoreType.DMA,
        ],
    )


def _sc_gather(feats, idx_flat):
    return _make_sc_gather()(feats, idx_flat)


S = 6
NSTEP = K // S


def _fsilu(x):
    return x / (1.0 + jnp.exp(-x))


def _layer_body(feats_ref, fj_ref, d_ref, w1_ref, b1_ref, w2_ref, b26_ref,
                wg6_ref, exp6_ref, bg_ref, lng_ref, lnb_ref, wn1_ref, bn1_ref,
                wn2_ref, bn2_ref, out_ref, p_scr, acc_scr):
    t = pl.program_id(0)

    @pl.when(t == 0)
    def _():
        p_scr[...] = (
            jnp.dot(feats_ref[...], w1_ref[0:DIM, :],
                    preferred_element_type=jnp.float32)
            + b1_ref[...]
        )
        acc_scr[...] = jnp.zeros((N, S * MDIM), jnp.float32)

    rowi = lax.broadcasted_iota(jnp.int32, (K, S), 0)
    colj = lax.broadcasted_iota(jnp.int32, (K, S), 1)
    onehot6 = (rowi == S * t + colj).astype(jnp.float32)
    d6 = jnp.dot(d_ref[...], onehot6, preferred_element_type=jnp.float32)

    p = p_scr[...]
    w1b = w1_ref[DIM : 2 * DIM, :]
    w1c = w1_ref[2 * DIM : 2 * DIM + 1, :]
    mks = []
    for s in range(S):
        qk = jnp.dot(fj_ref[s], w1b, preferred_element_type=jnp.float32)
        pre = p + qk + d6[:, s : s + 1] * w1c
        h = _fsilu(pre)
        mks.append(jnp.dot(h, w2_ref[...], preferred_element_type=jnp.float32))
    mk_cat = _fsilu(jnp.concatenate(mks, axis=1) + b26_ref[...])
    gpre = jnp.dot(mk_cat, wg6_ref[...], preferred_element_type=jnp.float32) \
        + bg_ref[...]
    g = 1.0 / (1.0 + jnp.exp(-gpre))
    gexp = jnp.dot(g, exp6_ref[...], preferred_element_type=jnp.float32)
    acc_scr[...] += mk_cat * gexp

    @pl.when(t == NSTEP - 1)
    def _():
        feats = feats_ref[...]
        acc = acc_scr[...]
        m_i = acc[:, 0:MDIM]
        for s in range(1, S):
            m_i = m_i + acc[:, s * MDIM : (s + 1) * MDIM]
        mu = jnp.mean(feats, axis=-1, keepdims=True)
        var = jnp.mean((feats - mu) ** 2, axis=-1, keepdims=True)
        normed = (feats - mu) / jnp.sqrt(var + 1e-5) * lng_ref[...] + lnb_ref[...]
        nh = _fsilu(
            jnp.dot(normed, wn1_ref[0:DIM, :], preferred_element_type=jnp.float32)
            + jnp.dot(m_i, wn1_ref[DIM : DIM + MDIM, :],
                      preferred_element_type=jnp.float32)
            + bn1_ref[...]
        )
        out_ref[...] = (
            jnp.dot(nh, wn2_ref[...], preferred_element_type=jnp.float32)
            + bn2_ref[...]
            + feats
        )


def _layer(feats, fj, d, w1, b1, w2, b2, wg, bg, lng, lnb, wn1, bn1, wn2, bn2):
    whole = lambda shape: pl.BlockSpec(shape, lambda t: tuple(0 for _ in shape))
    b26 = jnp.tile(b2.reshape(1, MDIM), (1, S))
    wg6 = jnp.kron(jnp.eye(S, dtype=jnp.float32), wg)
    exp6 = jnp.kron(jnp.eye(S, dtype=jnp.float32),
                    jnp.ones((1, MDIM), jnp.float32))
    return pl.pallas_call(
        _layer_body,
        grid=(NSTEP,),
        in_specs=[
            whole((N, DIM)),
            pl.BlockSpec((S, N, DIM), lambda t: (t, 0, 0)),
            whole((N, K)),
            whole((EI, HID)),
            whole((1, HID)),
            whole((HID, MDIM)),
            whole((1, S * MDIM)),
            whole((S * MDIM, S)),
            whole((S, S * MDIM)),
            whole((1, 1)),
            whole((1, DIM)),
            whole((1, DIM)),
            whole((DIM + MDIM, 2 * DIM)),
            whole((1, 2 * DIM)),
            whole((2 * DIM, DIM)),
            whole((1, DIM)),
        ],
        out_specs=whole((N, DIM)),
        out_shape=jax.ShapeDtypeStruct((N, DIM), jnp.float32),
        scratch_shapes=[
            pltpu.VMEM((N, HID), jnp.float32),
            pltpu.VMEM((N, S * MDIM), jnp.float32),
        ],
    )(feats, fj.reshape(K, N, DIM), d, w1, b1.reshape(1, HID),
      w2, b26, wg6, exp6, bg.reshape(1, 1),
      lng.reshape(1, DIM), lnb.reshape(1, DIM),
      wn1, bn1.reshape(1, 2 * DIM), wn2, bn2.reshape(1, DIM))


def kernel(feats, coords, W1, b1, W2, b2, Wg, bg, ln_g, ln_b, Wn1, bn1, Wn2, bn2):
    f = feats[0]
    c = coords[0]
    idx, d = _topk(c, c.T)
    idx_flat = idx.T.reshape(-1)
    for l in range(3):
        fj = _sc_gather(f, idx_flat)
        f = _layer(f, fj, d, W1[l], b1[l], W2[l], b2[l], Wg[l], bg[l],
                   ln_g[l], ln_b[l], Wn1[l], bn1[l], Wn2[l], bn2[l])
    return f[None]


# speedup vs baseline: 1.3673x; 1.0364x over previous
import functools

import jax
import jax.numpy as jnp
from jax import lax
from jax.experimental import pallas as pl
from jax.experimental.pallas import tpu as pltpu
from jax.experimental.pallas import tpu_sc as plsc

N = 2048
DIM = 128
K = 30
EI = 2 * DIM + 1
HID = 2 * EI
MDIM = 16
TOPK_BLK = 256
N_WORKERS = 32
ROWS_PER_W = (N * K) // N_WORKERS
GCH = 128
NCH = ROWS_PER_W // GCH


def _silu(x):
    return x * jax.nn.sigmoid(x)


def _topk_body(coords_ref, coords_t_ref, idx_ref, d_ref):
    dist = jnp.zeros((TOPK_BLK, N), jnp.float32)
    for c in range(3):
        col = coords_ref[:, c : c + 1]
        row = coords_t_ref[c : c + 1, :]
        diff = col - row
        dist = dist + diff * diff
    col_iota = lax.broadcasted_iota(jnp.int32, (TOPK_BLK, N), 1)
    sel_iota = lax.broadcasted_iota(jnp.int32, (TOPK_BLK, K), 1)

    def it(t, carry):
        dist, idx_acc, d_acc = carry
        mn = jnp.min(dist, axis=1, keepdims=True)
        am = jnp.min(jnp.where(dist == mn, col_iota, N), axis=1, keepdims=True)
        dist = jnp.where(col_iota == am, jnp.inf, dist)
        sel = sel_iota == t
        idx_acc = jnp.where(sel, am, idx_acc)
        d_acc = jnp.where(sel, mn, d_acc)
        return dist, idx_acc, d_acc

    _, idx_acc, d_acc = lax.fori_loop(
        0, K, it,
        (dist,
         jnp.zeros((TOPK_BLK, K), jnp.int32),
         jnp.zeros((TOPK_BLK, K), jnp.float32)),
        unroll=15,
    )
    idx_ref[...] = idx_acc
    d_ref[...] = d_acc


def _topk(coords, coords_t):
    return pl.pallas_call(
        _topk_body,
        grid=(N // TOPK_BLK,),
        in_specs=[
            pl.BlockSpec((TOPK_BLK, 3), lambda i: (i, 0)),
            pl.BlockSpec((3, N), lambda i: (0, 0)),
        ],
        out_specs=[
            pl.BlockSpec((TOPK_BLK, K), lambda i: (i, 0)),
            pl.BlockSpec((TOPK_BLK, K), lambda i: (i, 0)),
        ],
        out_shape=[
            jax.ShapeDtypeStruct((N, K), jnp.int32),
            jax.ShapeDtypeStruct((N, K), jnp.float32),
        ],
    )(coords, coords_t)


def _gather_body(feats_hbm, idx_hbm, out_hbm, idx_v, rows_v0, rows_v1,
                 gsem0, gsem1, ssem0, ssem1):
    wid = lax.axis_index("s") * 2 + lax.axis_index("c")
    base = wid * ROWS_PER_W
    pltpu.sync_copy(idx_hbm.at[pl.ds(base, ROWS_PER_W)], idx_v)

    bufs = (rows_v0, rows_v1)
    gsems = (gsem0, gsem1)
    ssems = (ssem0, ssem1)
    gcopies = [None, None]
    scopies = [None, None]
    for j in range(NCH):
        b = j & 1
        if scopies[b] is not None:
            scopies[b].wait()
        gcopies[b] = pltpu.async_copy(
            feats_hbm.at[idx_v.at[pl.ds(j * GCH, GCH)]], bufs[b], gsems[b]
        )
        if j >= 1:
            pb = (j - 1) & 1
            gcopies[pb].wait()
            scopies[pb] = pltpu.async_copy(
                bufs[pb], out_hbm.at[pl.ds(base + (j - 1) * GCH, GCH)],
                ssems[pb],
            )
    lb = (NCH - 1) & 1
    gcopies[lb].wait()
    scopies[lb] = pltpu.async_copy(
        bufs[lb], out_hbm.at[pl.ds(base + (NCH - 1) * GCH, GCH)], ssems[lb]
    )
    scopies[0].wait()
    scopies[1].wait()


@functools.cache
def _make_sc_gather():
    return pl.kernel(
        _gather_body,
        out_type=jax.ShapeDtypeStruct((N * K, DIM), jnp.float32),
        mesh=plsc.VectorSubcoreMesh(core_axis_name="c", subcore_axis_name="s"),
        scratch_types=[
            pltpu.VMEM((ROWS_PER_W,), jnp.int32),
            pltpu.VMEM((GCH, DIM), jnp.float32),
            pltpu.VMEM((GCH, DIM), jnp.float32),
            pltpu.SemaphoreType.DMA,
            pltpu.SemaphoreType.DMA,
            pltpu.SemaphoreType.DMA,
            pltpu.SemaphoreType.DMA,
        ],
    )


def _sc_gather(feats, idx_flat):
    return _make_sc_gather()(feats, idx_flat)


S = 6
NSTEP = K // S


def _fsilu(x):
    return x / (1.0 + jnp.exp(-x))


def _layer_body(feats_ref, fj_ref, d_ref, w1_ref, b1_ref, w2_ref, b26_ref,
                wg6_ref, exp6_ref, bg_ref, lng_ref, lnb_ref, wn1_ref, bn1_ref,
                wn2_ref, bn2_ref, out_ref, p_scr, acc_scr):
    t = pl.program_id(0)

    @pl.when(t == 0)
    def _():
        p_scr[...] = (
            jnp.dot(feats_ref[...], w1_ref[0:DIM, :],
                    preferred_element_type=jnp.float32)
            + b1_ref[...]
        )
        acc_scr[...] = jnp.zeros((N, S * MDIM), jnp.float32)

    rowi = lax.broadcasted_iota(jnp.int32, (K, S), 0)
    colj = lax.broadcasted_iota(jnp.int32, (K, S), 1)
    onehot6 = (rowi == S * t + colj).astype(jnp.float32)
    d6 = jnp.dot(d_ref[...], onehot6, preferred_element_type=jnp.float32)

    p = p_scr[...]
    w1b = w1_ref[DIM : 2 * DIM, :]
    w1c = w1_ref[2 * DIM : 2 * DIM + 1, :]
    mks = []
    for s in range(S):
        qk = jnp.dot(fj_ref[s], w1b, preferred_element_type=jnp.float32)
        pre = p + qk + d6[:, s : s + 1] * w1c
        h = _fsilu(pre)
        mks.append(jnp.dot(h, w2_ref[...], preferred_element_type=jnp.float32))
    mk_cat = _fsilu(jnp.concatenate(mks, axis=1) + b26_ref[...])
    gpre = jnp.dot(mk_cat, wg6_ref[...], preferred_element_type=jnp.float32) \
        + bg_ref[...]
    g = 1.0 / (1.0 + jnp.exp(-gpre))
    gexp = jnp.dot(g, exp6_ref[...], preferred_element_type=jnp.float32)
    acc_scr[...] += mk_cat * gexp

    @pl.when(t == NSTEP - 1)
    def _():
        feats = feats_ref[...]
        acc = acc_scr[...]
        m_i = acc[:, 0:MDIM]
        for s in range(1, S):
            m_i = m_i + acc[:, s * MDIM : (s + 1) * MDIM]
        mu = jnp.mean(feats, axis=-1, keepdims=True)
        var = jnp.mean((feats - mu) ** 2, axis=-1, keepdims=True)
        normed = (feats - mu) / jnp.sqrt(var + 1e-5) * lng_ref[...] + lnb_ref[...]
        nh = _fsilu(
            jnp.dot(normed, wn1_ref[0:DIM, :], preferred_element_type=jnp.float32)
            + jnp.dot(m_i, wn1_ref[DIM : DIM + MDIM, :],
                      preferred_element_type=jnp.float32)
            + bn1_ref[...]
        )
        out_ref[...] = (
            jnp.dot(nh, wn2_ref[...], preferred_element_type=jnp.float32)
            + bn2_ref[...]
            + feats
        )


def _layer(feats, fj, d, w1, b1, w2, b2, wg, bg, lng, lnb, wn1, bn1, wn2, bn2):
    whole = lambda shape: pl.BlockSpec(shape, lambda t: tuple(0 for _ in shape))
    b26 = jnp.tile(b2.reshape(1, MDIM), (1, S))
    wg6 = jnp.kron(jnp.eye(S, dtype=jnp.float32), wg)
    exp6 = jnp.kron(jnp.eye(S, dtype=jnp.float32),
                    jnp.ones((1, MDIM), jnp.float32))
    return pl.pallas_call(
        _layer_body,
        grid=(NSTEP,),
        in_specs=[
            whole((N, DIM)),
            pl.BlockSpec((S, N, DIM), lambda t: (t, 0, 0)),
            whole((N, K)),
            whole((EI, HID)),
            whole((1, HID)),
            whole((HID, MDIM)),
            whole((1, S * MDIM)),
            whole((S * MDIM, S)),
            whole((S, S * MDIM)),
            whole((1, 1)),
            whole((1, DIM)),
            whole((1, DIM)),
            whole((DIM + MDIM, 2 * DIM)),
            whole((1, 2 * DIM)),
            whole((2 * DIM, DIM)),
            whole((1, DIM)),
        ],
        out_specs=whole((N, DIM)),
        out_shape=jax.ShapeDtypeStruct((N, DIM), jnp.float32),
        scratch_shapes=[
            pltpu.VMEM((N, HID), jnp.float32),
            pltpu.VMEM((N, S * MDIM), jnp.float32),
        ],
    )(feats, fj.reshape(K, N, DIM), d, w1, b1.reshape(1, HID),
      w2, b26, wg6, exp6, bg.reshape(1, 1),
      lng.reshape(1, DIM), lnb.reshape(1, DIM),
      wn1, bn1.reshape(1, 2 * DIM), wn2, bn2.reshape(1, DIM))


def kernel(feats, coords, W1, b1, W2, b2, Wg, bg, ln_g, ln_b, Wn1, bn1, Wn2, bn2):
    f = feats[0]
    c = coords[0]
    idx, d = _topk(c, c.T)
    idx_flat = idx.T.reshape(-1)
    for l in range(3):
        fj = _sc_gather(f, idx_flat)
        f = _layer(f, fj, d, W1[l], b1[l], W2[l], b2[l], Wg[l], bg[l],
                   ln_g[l], ln_b[l], Wn1[l], bn1[l], Wn2[l], bn2[l])
    return f[None]


# speedup vs baseline: 1.4327x; 1.0478x over previous
import functools

import jax
import jax.numpy as jnp
from jax import lax
from jax.experimental import pallas as pl
from jax.experimental.pallas import tpu as pltpu
from jax.experimental.pallas import tpu_sc as plsc

N = 2048
DIM = 128
K = 30
EI = 2 * DIM + 1
HID = 2 * EI
MDIM = 16
TOPK_BLK = 256
N_WORKERS = 32
ROWS_PER_W = (N * K) // N_WORKERS
GCH = 128
NCH = ROWS_PER_W // GCH


def _silu(x):
    return x * jax.nn.sigmoid(x)


def _topk_body(coords_ref, coords_t_ref, idx_ref, d_ref):
    dist = jnp.zeros((TOPK_BLK, N), jnp.float32)
    for c in range(3):
        col = coords_ref[:, c : c + 1]
        row = coords_t_ref[c : c + 1, :]
        diff = col - row
        dist = dist + diff * diff
    col_iota = lax.broadcasted_iota(jnp.int32, (TOPK_BLK, N), 1)
    sel_iota = lax.broadcasted_iota(jnp.int32, (TOPK_BLK, K), 1)

    def it(t, carry):
        dist, idx_acc, d_acc = carry
        mn = jnp.min(dist, axis=1, keepdims=True)
        am = jnp.min(jnp.where(dist == mn, col_iota, N), axis=1, keepdims=True)
        dist = jnp.where(col_iota == am, jnp.inf, dist)
        sel = sel_iota == t
        idx_acc = jnp.where(sel, am, idx_acc)
        d_acc = jnp.where(sel, mn, d_acc)
        return dist, idx_acc, d_acc

    _, idx_acc, d_acc = lax.fori_loop(
        0, K, it,
        (dist,
         jnp.zeros((TOPK_BLK, K), jnp.int32),
         jnp.zeros((TOPK_BLK, K), jnp.float32)),
        unroll=15,
    )
    idx_ref[...] = idx_acc
    d_ref[...] = d_acc


def _topk(coords, coords_t):
    return pl.pallas_call(
        _topk_body,
        grid=(N // TOPK_BLK,),
        in_specs=[
            pl.BlockSpec((TOPK_BLK, 3), lambda i: (i, 0)),
            pl.BlockSpec((3, N), lambda i: (0, 0)),
        ],
        out_specs=[
            pl.BlockSpec((TOPK_BLK, K), lambda i: (i, 0)),
            pl.BlockSpec((TOPK_BLK, K), lambda i: (i, 0)),
        ],
        out_shape=[
            jax.ShapeDtypeStruct((N, K), jnp.int32),
            jax.ShapeDtypeStruct((N, K), jnp.float32),
        ],
    )(coords, coords_t)


def _gather_body(feats_hbm, idx_hbm, out_hbm, idx_v, rows_v0, rows_v1,
                 gsem0, gsem1, ssem0, ssem1):
    wid = lax.axis_index("s") * 2 + lax.axis_index("c")
    base = wid * ROWS_PER_W
    pltpu.sync_copy(idx_hbm.at[pl.ds(base, ROWS_PER_W)], idx_v)

    bufs = (rows_v0, rows_v1)
    gsems = (gsem0, gsem1)
    ssems = (ssem0, ssem1)
    gcopies = [None, None]
    scopies = [None, None]
    for j in range(NCH):
        b = j & 1
        if scopies[b] is not None:
            scopies[b].wait()
        gcopies[b] = pltpu.async_copy(
            feats_hbm.at[idx_v.at[pl.ds(j * GCH, GCH)]], bufs[b], gsems[b]
        )
        if j >= 1:
            pb = (j - 1) & 1
            gcopies[pb].wait()
            scopies[pb] = pltpu.async_copy(
                bufs[pb], out_hbm.at[pl.ds(base + (j - 1) * GCH, GCH)],
                ssems[pb],
            )
    lb = (NCH - 1) & 1
    gcopies[lb].wait()
    scopies[lb] = pltpu.async_copy(
        bufs[lb], out_hbm.at[pl.ds(base + (NCH - 1) * GCH, GCH)], ssems[lb]
    )
    scopies[0].wait()
    scopies[1].wait()


@functools.cache
def _make_sc_gather():
    return pl.kernel(
        _gather_body,
        out_type=jax.ShapeDtypeStruct((N * K, DIM), jnp.float32),
        mesh=plsc.VectorSubcoreMesh(core_axis_name="c", subcore_axis_name="s"),
        scratch_types=[
            pltpu.VMEM((ROWS_PER_W,), jnp.int32),
            pltpu.VMEM((GCH, DIM), jnp.float32),
            pltpu.VMEM((GCH, DIM), jnp.float32),
            pltpu.SemaphoreType.DMA,
            pltpu.SemaphoreType.DMA,
            pltpu.SemaphoreType.DMA,
            pltpu.SemaphoreType.DMA,
        ],
    )


def _sc_gather(feats, idx_flat):
    return _make_sc_gather()(feats, idx_flat)


S = 6
NSTEP = K // S


def _fsilu(x):
    return x / (1.0 + jnp.exp(-x))


def _layer_body(feats_ref, fj_ref, d_ref, w1_ref, b1_ref, w2_ref, b26_ref,
                wg6_ref, exp6_ref, bg_ref, lng_ref, lnb_ref, wn1_ref, bn1_ref,
                wn2_ref, bn2_ref, out_ref, p_scr, acc_scr):
    t = pl.program_id(0)

    @pl.when(t == 0)
    def _():
        p_scr[...] = (
            jnp.dot(feats_ref[...], w1_ref[0:DIM, :],
                    preferred_element_type=jnp.float32)
            + b1_ref[...]
        )
        acc_scr[...] = jnp.zeros((N, S * MDIM), jnp.float32)

    rowi = lax.broadcasted_iota(jnp.int32, (K, S), 0)
    colj = lax.broadcasted_iota(jnp.int32, (K, S), 1)
    onehot6 = (rowi == S * t + colj).astype(jnp.float32)
    d6 = jnp.dot(d_ref[...], onehot6, preferred_element_type=jnp.float32)

    p = p_scr[...]
    w1b = w1_ref[DIM : 2 * DIM, :]
    w1c = w1_ref[2 * DIM : 2 * DIM + 1, :]
    mks = []
    w2b = w2_ref[...].astype(jnp.bfloat16)
    for s in range(S):
        qk = jnp.dot(fj_ref[s], w1b, preferred_element_type=jnp.float32)
        pre = (p + qk + d6[:, s : s + 1] * w1c).astype(jnp.bfloat16)
        h = _fsilu(pre)
        mks.append(jnp.dot(h, w2b, preferred_element_type=jnp.float32))
    mk_cat = _fsilu(jnp.concatenate(mks, axis=1) + b26_ref[...])
    gpre = jnp.dot(mk_cat, wg6_ref[...], preferred_element_type=jnp.float32) \
        + bg_ref[...]
    g = 1.0 / (1.0 + jnp.exp(-gpre))
    gexp = jnp.dot(g, exp6_ref[...], preferred_element_type=jnp.float32)
    acc_scr[...] += mk_cat * gexp

    @pl.when(t == NSTEP - 1)
    def _():
        feats = feats_ref[...]
        acc = acc_scr[...]
        m_i = acc[:, 0:MDIM]
        for s in range(1, S):
            m_i = m_i + acc[:, s * MDIM : (s + 1) * MDIM]
        mu = jnp.mean(feats, axis=-1, keepdims=True)
        var = jnp.mean((feats - mu) ** 2, axis=-1, keepdims=True)
        normed = (feats - mu) / jnp.sqrt(var + 1e-5) * lng_ref[...] + lnb_ref[...]
        nh = _fsilu(
            jnp.dot(normed, wn1_ref[0:DIM, :], preferred_element_type=jnp.float32)
            + jnp.dot(m_i, wn1_ref[DIM : DIM + MDIM, :],
                      preferred_element_type=jnp.float32)
            + bn1_ref[...]
        )
        out_ref[...] = (
            jnp.dot(nh, wn2_ref[...], preferred_element_type=jnp.float32)
            + bn2_ref[...]
            + feats
        )


def _layer(feats, fj, d, w1, b1, w2, b2, wg, bg, lng, lnb, wn1, bn1, wn2, bn2):
    whole = lambda shape: pl.BlockSpec(shape, lambda t: tuple(0 for _ in shape))
    b26 = jnp.tile(b2.reshape(1, MDIM), (1, S))
    wg6 = jnp.kron(jnp.eye(S, dtype=jnp.float32), wg)
    exp6 = jnp.kron(jnp.eye(S, dtype=jnp.float32),
                    jnp.ones((1, MDIM), jnp.float32))
    return pl.pallas_call(
        _layer_body,
        grid=(NSTEP,),
        in_specs=[
            whole((N, DIM)),
            pl.BlockSpec((S, N, DIM), lambda t: (t, 0, 0)),
            whole((N, K)),
            whole((EI, HID)),
            whole((1, HID)),
            whole((HID, MDIM)),
            whole((1, S * MDIM)),
            whole((S * MDIM, S)),
            whole((S, S * MDIM)),
            whole((1, 1)),
            whole((1, DIM)),
            whole((1, DIM)),
            whole((DIM + MDIM, 2 * DIM)),
            whole((1, 2 * DIM)),
            whole((2 * DIM, DIM)),
            whole((1, DIM)),
        ],
        out_specs=whole((N, DIM)),
        out_shape=jax.ShapeDtypeStruct((N, DIM), jnp.float32),
        scratch_shapes=[
            pltpu.VMEM((N, HID), jnp.float32),
            pltpu.VMEM((N, S * MDIM), jnp.float32),
        ],
    )(feats, fj.reshape(K, N, DIM), d, w1, b1.reshape(1, HID),
      w2, b26, wg6, exp6, bg.reshape(1, 1),
      lng.reshape(1, DIM), lnb.reshape(1, DIM),
      wn1, bn1.reshape(1, 2 * DIM), wn2, bn2.reshape(1, DIM))


def kernel(feats, coords, W1, b1, W2, b2, Wg, bg, ln_g, ln_b, Wn1, bn1, Wn2, bn2):
    f = feats[0]
    c = coords[0]
    idx, d = _topk(c, c.T)
    idx_flat = idx.T.reshape(-1)
    for l in range(3):
        fj = _sc_gather(f, idx_flat)
        f = _layer(f, fj, d, W1[l], b1[l], W2[l], b2[l], Wg[l], bg[l],
                   ln_g[l], ln_b[l], Wn1[l], bn1[l], Wn2[l], bn2[l])
    return f[None]


# speedup vs baseline: 1.4430x; 1.0072x over previous
import functools

import jax
import jax.numpy as jnp
from jax import lax
from jax.experimental import pallas as pl
from jax.experimental.pallas import tpu as pltpu
from jax.experimental.pallas import tpu_sc as plsc

N = 2048
DIM = 128
K = 30
EI = 2 * DIM + 1
HID = 2 * EI
MDIM = 16
TOPK_BLK = 256
N_WORKERS = 32
ROWS_PER_W = (N * K) // N_WORKERS
GCH = 128
NCH = ROWS_PER_W // GCH


def _silu(x):
    return x * jax.nn.sigmoid(x)


def _topk_body(coords_ref, coords_t_ref, idx_ref, d_ref):
    dist = jnp.zeros((TOPK_BLK, N), jnp.float32)
    for c in range(3):
        col = coords_ref[:, c : c + 1]
        row = coords_t_ref[c : c + 1, :]
        diff = col - row
        dist = dist + diff * diff
    col_iota = lax.broadcasted_iota(jnp.int32, (TOPK_BLK, N), 1)
    sel_iota = lax.broadcasted_iota(jnp.int32, (TOPK_BLK, K), 1)

    def it(t, carry):
        dist, idx_acc, d_acc = carry
        mn = jnp.min(dist, axis=1, keepdims=True)
        am = jnp.min(jnp.where(dist == mn, col_iota, N), axis=1, keepdims=True)
        dist = jnp.where(col_iota == am, jnp.inf, dist)
        sel = sel_iota == t
        idx_acc = jnp.where(sel, am, idx_acc)
        d_acc = jnp.where(sel, mn, d_acc)
        return dist, idx_acc, d_acc

    _, idx_acc, d_acc = lax.fori_loop(
        0, K, it,
        (dist,
         jnp.zeros((TOPK_BLK, K), jnp.int32),
         jnp.zeros((TOPK_BLK, K), jnp.float32)),
        unroll=15,
    )
    idx_ref[...] = idx_acc
    d_ref[...] = d_acc


def _topk(coords, coords_t):
    return pl.pallas_call(
        _topk_body,
        grid=(N // TOPK_BLK,),
        in_specs=[
            pl.BlockSpec((TOPK_BLK, 3), lambda i: (i, 0)),
            pl.BlockSpec((3, N), lambda i: (0, 0)),
        ],
        out_specs=[
            pl.BlockSpec((TOPK_BLK, K), lambda i: (i, 0)),
            pl.BlockSpec((TOPK_BLK, K), lambda i: (i, 0)),
        ],
        out_shape=[
            jax.ShapeDtypeStruct((N, K), jnp.int32),
            jax.ShapeDtypeStruct((N, K), jnp.float32),
        ],
    )(coords, coords_t)


def _gather_body(feats_hbm, idx_hbm, out_hbm, idx_v, rows_v0, rows_v1,
                 gsem0, gsem1, ssem0, ssem1):
    wid = lax.axis_index("s") * 2 + lax.axis_index("c")
    base = wid * ROWS_PER_W
    pltpu.sync_copy(idx_hbm.at[pl.ds(base, ROWS_PER_W)], idx_v)

    bufs = (rows_v0, rows_v1)
    gsems = (gsem0, gsem1)
    ssems = (ssem0, ssem1)
    gcopies = [None, None]
    scopies = [None, None]
    for j in range(NCH):
        b = j & 1
        if scopies[b] is not None:
            scopies[b].wait()
        gcopies[b] = pltpu.async_copy(
            feats_hbm.at[idx_v.at[pl.ds(j * GCH, GCH)]], bufs[b], gsems[b]
        )
        if j >= 1:
            pb = (j - 1) & 1
            gcopies[pb].wait()
            scopies[pb] = pltpu.async_copy(
                bufs[pb], out_hbm.at[pl.ds(base + (j - 1) * GCH, GCH)],
                ssems[pb],
            )
    lb = (NCH - 1) & 1
    gcopies[lb].wait()
    scopies[lb] = pltpu.async_copy(
        bufs[lb], out_hbm.at[pl.ds(base + (NCH - 1) * GCH, GCH)], ssems[lb]
    )
    scopies[0].wait()
    scopies[1].wait()


@functools.cache
def _make_sc_gather():
    return pl.kernel(
        _gather_body,
        out_type=jax.ShapeDtypeStruct((N * K, DIM), jnp.float32),
        mesh=plsc.VectorSubcoreMesh(core_axis_name="c", subcore_axis_name="s"),
        scratch_types=[
            pltpu.VMEM((ROWS_PER_W,), jnp.int32),
            pltpu.VMEM((GCH, DIM), jnp.float32),
            pltpu.VMEM((GCH, DIM), jnp.float32),
            pltpu.SemaphoreType.DMA,
            pltpu.SemaphoreType.DMA,
            pltpu.SemaphoreType.DMA,
            pltpu.SemaphoreType.DMA,
        ],
    )


def _sc_gather(feats, idx_flat):
    return _make_sc_gather()(feats, idx_flat)


S = 6
NSTEP = K // S


def _fsilu(x):
    return x / (1.0 + jnp.exp(-x))


def _layer_body(feats_ref, fj_ref, d_ref, w1_ref, w1bb_ref, b1_ref, w2_ref, b26_ref,
                wg6_ref, exp6_ref, bg_ref, lng_ref, lnb_ref, wn1_ref, bn1_ref,
                wn2_ref, bn2_ref, out_ref, p_scr, acc_scr):
    t = pl.program_id(0)

    @pl.when(t == 0)
    def _():
        p_scr[...] = (
            jnp.dot(feats_ref[...], w1_ref[0:DIM, :],
                    preferred_element_type=jnp.float32)
            + b1_ref[...]
        )
        acc_scr[...] = jnp.zeros((N, S * MDIM), jnp.float32)

    rowi = lax.broadcasted_iota(jnp.int32, (K, S), 0)
    colj = lax.broadcasted_iota(jnp.int32, (K, S), 1)
    onehot6 = (rowi == S * t + colj).astype(jnp.float32)
    d6 = jnp.dot(d_ref[...], onehot6, preferred_element_type=jnp.float32)

    p = p_scr[...]
    w1c = w1_ref[2 * DIM : 2 * DIM + 1, :]
    mks = []
    w2b = w2_ref[...].astype(jnp.bfloat16)
    for s in range(S):
        qk = jnp.dot(fj_ref[s].astype(jnp.bfloat16), w1bb_ref[...],
                     preferred_element_type=jnp.float32)
        pre = (p + qk + d6[:, s : s + 1] * w1c).astype(jnp.bfloat16)
        h = _fsilu(pre)
        mks.append(jnp.dot(h, w2b, preferred_element_type=jnp.float32))
    mk_cat = _fsilu(jnp.concatenate(mks, axis=1) + b26_ref[...])
    gpre = jnp.dot(mk_cat, wg6_ref[...], preferred_element_type=jnp.float32) \
        + bg_ref[...]
    g = 1.0 / (1.0 + jnp.exp(-gpre))
    gexp = jnp.dot(g, exp6_ref[...], preferred_element_type=jnp.float32)
    acc_scr[...] += mk_cat * gexp

    @pl.when(t == NSTEP - 1)
    def _():
        feats = feats_ref[...]
        acc = acc_scr[...]
        m_i = acc[:, 0:MDIM]
        for s in range(1, S):
            m_i = m_i + acc[:, s * MDIM : (s + 1) * MDIM]
        mu = jnp.mean(feats, axis=-1, keepdims=True)
        var = jnp.mean((feats - mu) ** 2, axis=-1, keepdims=True)
        normed = (feats - mu) / jnp.sqrt(var + 1e-5) * lng_ref[...] + lnb_ref[...]
        nh = _fsilu(
            jnp.dot(normed, wn1_ref[0:DIM, :], preferred_element_type=jnp.float32)
            + jnp.dot(m_i, wn1_ref[DIM : DIM + MDIM, :],
                      preferred_element_type=jnp.float32)
            + bn1_ref[...]
        )
        out_ref[...] = (
            jnp.dot(nh, wn2_ref[...], preferred_element_type=jnp.float32)
            + bn2_ref[...]
            + feats
        )


def _layer(feats, fj, d, w1, b1, w2, b2, wg, bg, lng, lnb, wn1, bn1, wn2, bn2):
    whole = lambda shape: pl.BlockSpec(shape, lambda t: tuple(0 for _ in shape))
    b26 = jnp.tile(b2.reshape(1, MDIM), (1, S))
    wg6 = jnp.kron(jnp.eye(S, dtype=jnp.float32), wg)
    exp6 = jnp.kron(jnp.eye(S, dtype=jnp.float32),
                    jnp.ones((1, MDIM), jnp.float32))
    return pl.pallas_call(
        _layer_body,
        grid=(NSTEP,),
        in_specs=[
            whole((N, DIM)),
            pl.BlockSpec((S, N, DIM), lambda t: (t, 0, 0)),
            whole((N, K)),
            whole((EI, HID)),
            whole((DIM, HID)),
            whole((1, HID)),
            whole((HID, MDIM)),
            whole((1, S * MDIM)),
            whole((S * MDIM, S)),
            whole((S, S * MDIM)),
            whole((1, 1)),
            whole((1, DIM)),
            whole((1, DIM)),
            whole((DIM + MDIM, 2 * DIM)),
            whole((1, 2 * DIM)),
            whole((2 * DIM, DIM)),
            whole((1, DIM)),
        ],
        out_specs=whole((N, DIM)),
        out_shape=jax.ShapeDtypeStruct((N, DIM), jnp.float32),
        scratch_shapes=[
            pltpu.VMEM((N, HID), jnp.float32),
            pltpu.VMEM((N, S * MDIM), jnp.float32),
        ],
    )(feats, fj.reshape(K, N, DIM), d, w1,
      w1[DIM : 2 * DIM].astype(jnp.bfloat16), b1.reshape(1, HID),
      w2, b26, wg6, exp6, bg.reshape(1, 1),
      lng.reshape(1, DIM), lnb.reshape(1, DIM),
      wn1, bn1.reshape(1, 2 * DIM), wn2, bn2.reshape(1, DIM))


def kernel(feats, coords, W1, b1, W2, b2, Wg, bg, ln_g, ln_b, Wn1, bn1, Wn2, bn2):
    f = feats[0]
    c = coords[0]
    idx, d = _topk(c, c.T)
    idx_flat = idx.T.reshape(-1)
    for l in range(3):
        fj = _sc_gather(f, idx_flat)
        f = _layer(f, fj, d, W1[l], b1[l], W2[l], b2[l], Wg[l], bg[l],
                   ln_g[l], ln_b[l], Wn1[l], bn1[l], Wn2[l], bn2[l])
    return f[None]


# speedup vs baseline: 1.5035x; 1.0419x over previous
import functools

import jax
import jax.numpy as jnp
from jax import lax
from jax.experimental import pallas as pl
from jax.experimental.pallas import tpu as pltpu
from jax.experimental.pallas import tpu_sc as plsc

N = 2048
DIM = 128
K = 30
EI = 2 * DIM + 1
HID = 2 * EI
MDIM = 16
TOPK_BLK = 256
N_WORKERS = 32
ROWS_PER_W = (N * K) // N_WORKERS
GCH = 128
NCH = ROWS_PER_W // GCH


def _silu(x):
    return x * jax.nn.sigmoid(x)


def _topk_body(coords_ref, coords_t_ref, idx_ref, d_ref):
    dist = jnp.zeros((TOPK_BLK, N), jnp.float32)
    for c in range(3):
        col = coords_ref[:, c : c + 1]
        row = coords_t_ref[c : c + 1, :]
        diff = col - row
        dist = dist + diff * diff
    col_iota = lax.broadcasted_iota(jnp.int32, (TOPK_BLK, N), 1)
    sel_iota = lax.broadcasted_iota(jnp.int32, (TOPK_BLK, K), 1)

    def it(t, carry):
        dist, idx_acc, d_acc = carry
        mn = jnp.min(dist, axis=1, keepdims=True)
        am = jnp.min(jnp.where(dist == mn, col_iota, N), axis=1, keepdims=True)
        dist = jnp.where(col_iota == am, jnp.inf, dist)
        sel = sel_iota == t
        idx_acc = jnp.where(sel, am, idx_acc)
        d_acc = jnp.where(sel, mn, d_acc)
        return dist, idx_acc, d_acc

    _, idx_acc, d_acc = lax.fori_loop(
        0, K, it,
        (dist,
         jnp.zeros((TOPK_BLK, K), jnp.int32),
         jnp.zeros((TOPK_BLK, K), jnp.float32)),
        unroll=30,
    )
    idx_ref[...] = idx_acc
    d_ref[...] = d_acc


def _topk(coords, coords_t):
    return pl.pallas_call(
        _topk_body,
        grid=(N // TOPK_BLK,),
        in_specs=[
            pl.BlockSpec((TOPK_BLK, 3), lambda i: (i, 0)),
            pl.BlockSpec((3, N), lambda i: (0, 0)),
        ],
        out_specs=[
            pl.BlockSpec((TOPK_BLK, K), lambda i: (i, 0)),
            pl.BlockSpec((TOPK_BLK, K), lambda i: (i, 0)),
        ],
        out_shape=[
            jax.ShapeDtypeStruct((N, K), jnp.int32),
            jax.ShapeDtypeStruct((N, K), jnp.float32),
        ],
    )(coords, coords_t)


def _gather_body(feats_hbm, idx_hbm, out_hbm, idx_v, rows_v0, rows_v1,
                 gsem0, gsem1, ssem0, ssem1):
    wid = lax.axis_index("s") * 2 + lax.axis_index("c")
    base = wid * ROWS_PER_W
    pltpu.sync_copy(idx_hbm.at[pl.ds(base, ROWS_PER_W)], idx_v)

    bufs = (rows_v0, rows_v1)
    gsems = (gsem0, gsem1)
    ssems = (ssem0, ssem1)
    gcopies = [None, None]
    scopies = [None, None]
    for j in range(NCH):
        b = j & 1
        if scopies[b] is not None:
            scopies[b].wait()
        gcopies[b] = pltpu.async_copy(
            feats_hbm.at[idx_v.at[pl.ds(j * GCH, GCH)]], bufs[b], gsems[b]
        )
        if j >= 1:
            pb = (j - 1) & 1
            gcopies[pb].wait()
            scopies[pb] = pltpu.async_copy(
                bufs[pb], out_hbm.at[pl.ds(base + (j - 1) * GCH, GCH)],
                ssems[pb],
            )
    lb = (NCH - 1) & 1
    gcopies[lb].wait()
    scopies[lb] = pltpu.async_copy(
        bufs[lb], out_hbm.at[pl.ds(base + (NCH - 1) * GCH, GCH)], ssems[lb]
    )
    scopies[0].wait()
    scopies[1].wait()


@functools.cache
def _make_sc_gather():
    return pl.kernel(
        _gather_body,
        out_type=jax.ShapeDtypeStruct((N * K, DIM), jnp.float32),
        mesh=plsc.VectorSubcoreMesh(core_axis_name="c", subcore_axis_name="s"),
        scratch_types=[
            pltpu.VMEM((ROWS_PER_W,), jnp.int32),
            pltpu.VMEM((GCH, DIM), jnp.float32),
            pltpu.VMEM((GCH, DIM), jnp.float32),
            pltpu.SemaphoreType.DMA,
            pltpu.SemaphoreType.DMA,
            pltpu.SemaphoreType.DMA,
            pltpu.SemaphoreType.DMA,
        ],
    )


def _sc_gather(feats, idx_flat):
    return _make_sc_gather()(feats, idx_flat)


S = 6
NSTEP = K // S


def _fsilu(x):
    return x / (1.0 + jnp.exp(-x))


def _layer_body(feats_ref, fj_ref, d_ref, w1_ref, w1bb_ref, b1_ref, w2_ref, b26_ref,
                wg6_ref, exp6_ref, bg_ref, lng_ref, lnb_ref, wn1_ref, bn1_ref,
                wn2_ref, bn2_ref, out_ref, p_scr, acc_scr):
    t = pl.program_id(0)

    @pl.when(t == 0)
    def _():
        p_scr[...] = (
            jnp.dot(feats_ref[...], w1_ref[0:DIM, :],
                    preferred_element_type=jnp.float32)
            + b1_ref[...]
        )
        acc_scr[...] = jnp.zeros((N, S * MDIM), jnp.float32)

    rowi = lax.broadcasted_iota(jnp.int32, (K, S), 0)
    colj = lax.broadcasted_iota(jnp.int32, (K, S), 1)
    onehot6 = (rowi == S * t + colj).astype(jnp.float32)
    d6 = jnp.dot(d_ref[...], onehot6, preferred_element_type=jnp.float32)

    p = p_scr[...]
    w1c = w1_ref[2 * DIM : 2 * DIM + 1, :]
    mks = []
    w2b = w2_ref[...].astype(jnp.bfloat16)
    for s in range(S):
        qk = jnp.dot(fj_ref[s].astype(jnp.bfloat16), w1bb_ref[...],
                     preferred_element_type=jnp.float32)
        pre = (p + qk + d6[:, s : s + 1] * w1c).astype(jnp.bfloat16)
        h = _fsilu(pre)
        mks.append(jnp.dot(h, w2b, preferred_element_type=jnp.float32))
    mk_cat = _fsilu(jnp.concatenate(mks, axis=1) + b26_ref[...])
    gpre = jnp.dot(mk_cat, wg6_ref[...], preferred_element_type=jnp.float32) \
        + bg_ref[...]
    g = 1.0 / (1.0 + jnp.exp(-gpre))
    gexp = jnp.dot(g, exp6_ref[...], preferred_element_type=jnp.float32)
    acc_scr[...] += mk_cat * gexp

    @pl.when(t == NSTEP - 1)
    def _():
        feats = feats_ref[...]
        acc = acc_scr[...]
        m_i = acc[:, 0:MDIM]
        for s in range(1, S):
            m_i = m_i + acc[:, s * MDIM : (s + 1) * MDIM]
        mu = jnp.mean(feats, axis=-1, keepdims=True)
        var = jnp.mean((feats - mu) ** 2, axis=-1, keepdims=True)
        normed = (feats - mu) / jnp.sqrt(var + 1e-5) * lng_ref[...] + lnb_ref[...]
        nh = _fsilu(
            jnp.dot(normed, wn1_ref[0:DIM, :], preferred_element_type=jnp.float32)
            + jnp.dot(m_i, wn1_ref[DIM : DIM + MDIM, :],
                      preferred_element_type=jnp.float32)
            + bn1_ref[...]
        )
        out_ref[...] = (
            jnp.dot(nh, wn2_ref[...], preferred_element_type=jnp.float32)
            + bn2_ref[...]
            + feats
        )


def _layer(feats, fj, d, w1, b1, w2, b2, wg, bg, lng, lnb, wn1, bn1, wn2, bn2):
    whole = lambda shape: pl.BlockSpec(shape, lambda t: tuple(0 for _ in shape))
    b26 = jnp.tile(b2.reshape(1, MDIM), (1, S))
    wg6 = jnp.kron(jnp.eye(S, dtype=jnp.float32), wg)
    exp6 = jnp.kron(jnp.eye(S, dtype=jnp.float32),
                    jnp.ones((1, MDIM), jnp.float32))
    return pl.pallas_call(
        _layer_body,
        grid=(NSTEP,),
        in_specs=[
            whole((N, DIM)),
            pl.BlockSpec((S, N, DIM), lambda t: (t, 0, 0)),
            whole((N, K)),
            whole((EI, HID)),
            whole((DIM, HID)),
            whole((1, HID)),
            whole((HID, MDIM)),
            whole((1, S * MDIM)),
            whole((S * MDIM, S)),
            whole((S, S * MDIM)),
            whole((1, 1)),
            whole((1, DIM)),
            whole((1, DIM)),
            whole((DIM + MDIM, 2 * DIM)),
            whole((1, 2 * DIM)),
            whole((2 * DIM, DIM)),
            whole((1, DIM)),
        ],
        out_specs=whole((N, DIM)),
        out_shape=jax.ShapeDtypeStruct((N, DIM), jnp.float32),
        scratch_shapes=[
            pltpu.VMEM((N, HID), jnp.float32),
            pltpu.VMEM((N, S * MDIM), jnp.float32),
        ],
    )(feats, fj.reshape(K, N, DIM), d, w1,
      w1[DIM : 2 * DIM].astype(jnp.bfloat16), b1.reshape(1, HID),
      w2, b26, wg6, exp6, bg.reshape(1, 1),
      lng.reshape(1, DIM), lnb.reshape(1, DIM),
      wn1, bn1.reshape(1, 2 * DIM), wn2, bn2.reshape(1, DIM))


def kernel(feats, coords, W1, b1, W2, b2, Wg, bg, ln_g, ln_b, Wn1, bn1, Wn2, bn2):
    f = feats[0]
    c = coords[0]
    idx, d = _topk(c, c.T)
    idx_flat = idx.T.reshape(-1)
    for l in range(3):
        fj = _sc_gather(f, idx_flat)
        f = _layer(f, fj, d, W1[l], b1[l], W2[l], b2[l], Wg[l], bg[l],
                   ln_g[l], ln_b[l], Wn1[l], bn1[l], Wn2[l], bn2[l])
    return f[None]
